# initial kernel scaffold (unmeasured)
import jax
import jax.numpy as jnp
from jax import lax
from jax.experimental import pallas as pl
from jax.experimental.pallas import tpu as pltpu


def kernel(
    x,
):
    def body(*refs):
        pass

    out_shape = jax.ShapeDtypeStruct(..., jnp.float32)
    return pl.pallas_call(body, out_shape=out_shape)(...)


# baseline (device time: 116550 ns/iter reference)
import jax
import jax.numpy as jnp
from jax import lax
from jax.experimental import pallas as pl
from jax.experimental.pallas import tpu as pltpu

N_X = 2


def kernel(x):
    _, m, n2 = x.shape
    n = n2 // N_X

    def body(x_ref, out_ref, send_buf, peer_buf, send_sem, recv_sem):
        my_x = lax.axis_index("x")
        my_y = lax.axis_index("y")
        nbr_x = (1 - my_x, my_y)

        barrier_sem = pltpu.get_barrier_semaphore()
        pl.semaphore_signal(
            barrier_sem, inc=1,
            device_id=nbr_x, device_id_type=pl.DeviceIdType.MESH,
        )
        pl.semaphore_wait(barrier_sem, 1)

        send_buf[...] = x_ref[0, :, pl.ds((1 - my_x) * n, n)].astype(
            jnp.bfloat16
        )

        rdma = pltpu.make_async_remote_copy(
            src_ref=send_buf,
            dst_ref=peer_buf,
            send_sem=send_sem,
            recv_sem=recv_sem,
            device_id=nbr_x,
            device_id_type=pl.DeviceIdType.MESH,
        )
        rdma.start()
        rdma.wait()

        out_ref[...] = (
            x_ref[0, :, pl.ds(my_x * n, n)].astype(jnp.bfloat16)
            + peer_buf[...]
        )

    return pl.pallas_call(
        body,
        out_shape=jax.ShapeDtypeStruct((m, n), jnp.bfloat16),
        in_specs=[pl.BlockSpec(memory_space=pltpu.VMEM)],
        out_specs=pl.BlockSpec(memory_space=pltpu.VMEM),
        scratch_shapes=[
            pltpu.VMEM((m, n), jnp.bfloat16),
            pltpu.VMEM((m, n), jnp.bfloat16),
            pltpu.SemaphoreType.DMA,
            pltpu.SemaphoreType.DMA,
        ],
        compiler_params=pltpu.CompilerParams(
            collective_id=0, vmem_limit_bytes=100 * 1024 * 1024
        ),
    )(x)


# device time: 77414 ns/iter; 1.5055x vs baseline; 1.5055x over previous
import jax
import jax.numpy as jnp
from jax import lax
from jax.experimental import pallas as pl
from jax.experimental.pallas import tpu as pltpu

N_X = 2
C = 16


def kernel(x):
    _, m, n2 = x.shape
    n = n2 // N_X
    half = m // 2
    ch = half // C

    def body(x_ref, out_ref, send_x_buf, peer_buf,
             sx_sems, rx_sems, sy_sems, ry_sems):
        my_x = lax.axis_index("x")
        my_y = lax.axis_index("y")
        nbr_x = (1 - my_x, my_y)
        nbr_y = (my_x, 1 - my_y)
        r0 = my_y * half
        r1 = (1 - my_y) * half

        barrier_sem = pltpu.get_barrier_semaphore()
        for nbr in (nbr_x, nbr_y):
            pl.semaphore_signal(
                barrier_sem, inc=1,
                device_id=nbr, device_id_type=pl.DeviceIdType.MESH,
            )
        pl.semaphore_wait(barrier_sem, 2)

        send_x_buf[...] = x_ref[
            0, pl.ds(r0, half), pl.ds((1 - my_x) * n, n)
        ].astype(jnp.bfloat16)

        x_rdmas = []
        for c in range(C):
            rdma = pltpu.make_async_remote_copy(
                src_ref=send_x_buf.at[pl.ds(c * ch, ch), :],
                dst_ref=peer_buf.at[pl.ds(r0 + c * ch, ch), :],
                send_sem=sx_sems.at[c],
                recv_sem=rx_sems.at[c],
                device_id=nbr_x,
                device_id_type=pl.DeviceIdType.MESH,
            )
            rdma.start()
            x_rdmas.append(rdma)

        y_rdmas = []
        for c in range(C):
            x_rdmas[c].wait_recv()
            rdma = pltpu.make_async_remote_copy(
                src_ref=peer_buf.at[pl.ds(r0 + c * ch, ch), :],
                dst_ref=peer_buf.at[pl.ds(r0 + c * ch, ch), :],
                send_sem=sy_sems.at[c],
                recv_sem=ry_sems.at[c],
                device_id=nbr_y,
                device_id_type=pl.DeviceIdType.MESH,
            )
            rdma.start()
            y_rdmas.append(rdma)

        out_ref[pl.ds(r0, half), :] = (
            x_ref[0, pl.ds(r0, half), pl.ds(my_x * n, n)].astype(jnp.bfloat16)
            + peer_buf[pl.ds(r0, half), :]
        )

        for c in range(C):
            y_rdmas[c].wait_recv()

        out_ref[pl.ds(r1, half), :] = (
            x_ref[0, pl.ds(r1, half), pl.ds(my_x * n, n)].astype(jnp.bfloat16)
            + peer_buf[pl.ds(r1, half), :]
        )

        for c in range(C):
            x_rdmas[c].wait_send()
            y_rdmas[c].wait_send()

    return pl.pallas_call(
        body,
        out_shape=jax.ShapeDtypeStruct((m, n), jnp.bfloat16),
        in_specs=[pl.BlockSpec(memory_space=pltpu.VMEM)],
        out_specs=pl.BlockSpec(memory_space=pltpu.VMEM),
        scratch_shapes=[
            pltpu.VMEM((half, n), jnp.bfloat16),
            pltpu.VMEM((m, n), jnp.bfloat16),
            pltpu.SemaphoreType.DMA((C,)),
            pltpu.SemaphoreType.DMA((C,)),
            pltpu.SemaphoreType.DMA((C,)),
            pltpu.SemaphoreType.DMA((C,)),
        ],
        compiler_params=pltpu.CompilerParams(
            collective_id=0, vmem_limit_bytes=100 * 1024 * 1024
        ),
    )(x)


# device time: 67979 ns/iter; 1.7145x vs baseline; 1.1388x over previous
import jax
import jax.numpy as jnp
from jax import lax
from jax.experimental import pallas as pl
from jax.experimental.pallas import tpu as pltpu

N_X = 2
C = 16


def kernel(x):
    _, m, n2 = x.shape
    n = n2 // N_X
    half = m // 2
    ch = half // C

    def body(x_hbm, out_ref, own_buf, stage_f32, send_x_buf, peer_buf,
             own_sem, stage_sems, sx_sems, rx_sems, sy_sems, ry_sems):
        my_x = lax.axis_index("x")
        my_y = lax.axis_index("y")
        nbr_x = (1 - my_x, my_y)
        nbr_y = (my_x, 1 - my_y)
        r0 = my_y * half
        r1 = (1 - my_y) * half

        own_copy = pltpu.make_async_copy(
            x_hbm.at[0, :, pl.ds(my_x * n, n)], own_buf, own_sem
        )
        own_copy.start()

        stage_copies = []
        for c in range(C):
            cp = pltpu.make_async_copy(
                x_hbm.at[0, pl.ds(r0 + c * ch, ch), pl.ds((1 - my_x) * n, n)],
                stage_f32.at[pl.ds(c * ch, ch), :],
                stage_sems.at[c],
            )
            cp.start()
            stage_copies.append(cp)

        barrier_sem = pltpu.get_barrier_semaphore()
        for nbr in (nbr_x, nbr_y):
            pl.semaphore_signal(
                barrier_sem, inc=1,
                device_id=nbr, device_id_type=pl.DeviceIdType.MESH,
            )
        pl.semaphore_wait(barrier_sem, 2)

        x_rdmas = []
        for c in range(C):
            stage_copies[c].wait()
            send_x_buf[pl.ds(c * ch, ch), :] = stage_f32[
                pl.ds(c * ch, ch), :
            ].astype(jnp.bfloat16)
            rdma = pltpu.make_async_remote_copy(
                src_ref=send_x_buf.at[pl.ds(c * ch, ch), :],
                dst_ref=peer_buf.at[pl.ds(r0 + c * ch, ch), :],
                send_sem=sx_sems.at[c],
                recv_sem=rx_sems.at[c],
                device_id=nbr_x,
                device_id_type=pl.DeviceIdType.MESH,
            )
            rdma.start()
            x_rdmas.append(rdma)

        own_copy.wait()

        y_rdmas = []
        for c in range(C):
            x_rdmas[c].wait_recv()
            rdma = pltpu.make_async_remote_copy(
                src_ref=peer_buf.at[pl.ds(r0 + c * ch, ch), :],
                dst_ref=peer_buf.at[pl.ds(r0 + c * ch, ch), :],
                send_sem=sy_sems.at[c],
                recv_sem=ry_sems.at[c],
                device_id=nbr_y,
                device_id_type=pl.DeviceIdType.MESH,
            )
            rdma.start()
            y_rdmas.append(rdma)
            out_ref[pl.ds(r0 + c * ch, ch), :] = (
                own_buf[pl.ds(r0 + c * ch, ch), :].astype(jnp.bfloat16)
                + peer_buf[pl.ds(r0 + c * ch, ch), :]
            )

        for c in range(C):
            y_rdmas[c].wait_recv()
            out_ref[pl.ds(r1 + c * ch, ch), :] = (
                own_buf[pl.ds(r1 + c * ch, ch), :].astype(jnp.bfloat16)
                + peer_buf[pl.ds(r1 + c * ch, ch), :]
            )

        for c in range(C):
            x_rdmas[c].wait_send()
            y_rdmas[c].wait_send()

    return pl.pallas_call(
        body,
        out_shape=jax.ShapeDtypeStruct((m, n), jnp.bfloat16),
        in_specs=[pl.BlockSpec(memory_space=pl.ANY)],
        out_specs=pl.BlockSpec(memory_space=pltpu.VMEM),
        scratch_shapes=[
            pltpu.VMEM((m, n), jnp.float32),
            pltpu.VMEM((half, n), jnp.float32),
            pltpu.VMEM((half, n), jnp.bfloat16),
            pltpu.VMEM((m, n), jnp.bfloat16),
            pltpu.SemaphoreType.DMA,
            pltpu.SemaphoreType.DMA((C,)),
            pltpu.SemaphoreType.DMA((C,)),
            pltpu.SemaphoreType.DMA((C,)),
            pltpu.SemaphoreType.DMA((C,)),
            pltpu.SemaphoreType.DMA((C,)),
        ],
        compiler_params=pltpu.CompilerParams(
            collective_id=0, vmem_limit_bytes=100 * 1024 * 1024
        ),
    )(x)
